# baseline (speedup 1.0000x reference)
import jax
import jax.numpy as jnp
from jax import lax
from jax.experimental import pallas as pl
from jax.experimental.pallas import tpu as pltpu
from jax.experimental.pallas import tpu_sc as plsc

NG = 256
NN = 64
NR = NN + 1
NF = 9
H = 768
L = 16
NC = 2
NS = 16
NW = NC * NS

GPW = NG // NW
SUB = 8
ROWS = SUB * NF
KPG = 9
GSTRIDE = 592
SMALL = 16
CB = H // L


def _body(idx9, ind_f, outd_f, atom_w, in_w, out_w, tok,
          out3, deg3,
          xidx, din, dout, bufA, bufB, obA, obB, tokv,
          semA, semB, semOA, semOB):
    wid = lax.axis_index("s") * NC + lax.axis_index("c")

    pltpu.sync_copy(idx9.at[pl.ds(wid * GPW * GSTRIDE, GPW * GSTRIDE)], xidx)
    pltpu.sync_copy(ind_f.at[pl.ds(wid * GPW * NN, GPW * NN)], din)
    pltpu.sync_copy(outd_f.at[pl.ds(wid * GPW * NN, GPW * NN)], dout)
    pltpu.sync_copy(tok, tokv)

    def gsize(k):
        return ROWS if k < 8 else SMALL

    def fire(gi, k, buf, sem):
        n = gsize(k)
        pltpu.async_copy(
            atom_w.at[xidx.at[pl.ds(gi * GSTRIDE + k * ROWS, n)]],
            buf.at[pl.ds(0, n)], sem)

    def wait_g(gi, k, buf, sem):
        n = gsize(k)
        pltpu.make_async_copy(
            atom_w.at[xidx.at[pl.ds(gi * GSTRIDE + k * ROWS, n)]],
            buf.at[pl.ds(0, n)], sem).wait()

    def compute(k, buf, ob):
        nrows = SUB if k < 8 else 1

        def jbody(j, _):
            c = j * L
            for n in range(nrows):
                if k == 0 and n == 0:
                    ob[0, pl.ds(c, L)] = tokv[0, pl.ds(c, L)]
                else:
                    base = n * NF
                    acc = buf[base, pl.ds(c, L)]
                    for f in range(1, NF):
                        acc = acc + buf[base + f, pl.ds(c, L)]
                    ob[n, pl.ds(c, L)] = acc
            return 0

        lax.fori_loop(0, CB, jbody, 0)

    def store(gi, k, ob, sem):
        g = wid * GPW + gi
        if k < 8:
            pltpu.async_copy(ob, out3.at[g, pl.ds(k * SUB, SUB), :], sem)
        else:
            pltpu.async_copy(ob.at[pl.ds(0, 1)], out3.at[g, pl.ds(NN, 1), :],
                             sem)

    def drain_store(nrows, ob, sem):
        pltpu.make_async_copy(out3.at[0, pl.ds(0, nrows), :],
                              ob.at[pl.ds(0, nrows)], sem).wait()

    fire(0, 0, bufA, semA)

    def pair_body(gp, _):
        gi0 = gp * 2
        for m in range(2 * KPG):
            gi = gi0 + m // KPG
            k = m % KPG
            buf, sem, ob, osem = ((bufA, semA, obA, semOA) if m % 2 == 0
                                  else (bufB, semB, obB, semOB))
            nbuf, nsem = (bufB, semB) if m % 2 == 0 else (bufA, semA)
            wait_g(gi, k, buf, sem)
            if m == 2 * KPG - 1:
                @pl.when(gi0 + 2 < GPW)
                def _():
                    fire(gi0 + 2, 0, nbuf, nsem)
            elif k == KPG - 1:
                fire(gi + 1, 0, nbuf, nsem)
            else:
                fire(gi, k + 1, nbuf, nsem)
            prev_k = (m - 2) % (2 * KPG) % KPG
            prev_rows = SUB if prev_k < 8 else 1
            if m >= 2:
                drain_store(prev_rows, ob, osem)
            else:
                @pl.when(gp > 0)
                def _():
                    drain_store(prev_rows, ob, osem)
            compute(k, buf, ob)
            store(gi, k, ob, osem)
        return 0

    lax.fori_loop(0, GPW // 2, pair_body, 0)
    drain_store(SUB, obA, semOA)
    drain_store(1, obB, semOB)

    DW = 32
    NWAVE = GPW * NN // DW

    def dfire(w, buf, sem):
        pltpu.async_copy(in_w.at[din.at[pl.ds(w * DW, DW)]],
                         buf.at[pl.ds(0, DW)], sem)
        pltpu.async_copy(out_w.at[dout.at[pl.ds(w * DW, DW)]],
                         buf.at[pl.ds(DW, DW)], sem)

    def dwait(w, buf, sem):
        pltpu.make_async_copy(in_w.at[din.at[pl.ds(w * DW, DW)]],
                              buf.at[pl.ds(0, DW)], sem).wait()
        pltpu.make_async_copy(out_w.at[dout.at[pl.ds(w * DW, DW)]],
                              buf.at[pl.ds(DW, DW)], sem).wait()

    dfire(0, bufA, semA)

    def dpair(wp, _):
        for t in range(2):
            w = wp * 2 + t
            buf, sem = (bufA, semA) if t == 0 else (bufB, semB)
            nbuf, nsem = (bufB, semB) if t == 0 else (bufA, semA)
            dwait(w, buf, sem)
            if t == 0:
                dfire(w + 1, nbuf, nsem)
            else:
                @pl.when(wp + 1 < NWAVE // 2)
                def _():
                    dfire(w + 1, nbuf, nsem)
            for q in range(4):
                ob, osem = ((obA, semOA) if (t * 4 + q) % 2 == 0
                            else (obB, semOB))
                if t == 0 and q < 2:
                    @pl.when(wp > 0)
                    def _():
                        drain_store(SUB, ob, osem)
                else:
                    drain_store(SUB, ob, osem)

                def jbody(j, _):
                    c = j * L
                    for n in range(SUB):
                        r = q * SUB + n
                        ob[n, pl.ds(c, L)] = (buf[r, pl.ds(c, L)]
                                              + buf[DW + r, pl.ds(c, L)])
                    return 0

                lax.fori_loop(0, CB, jbody, 0)
                g = wid * GPW + w // 2
                pltpu.async_copy(
                    ob, deg3.at[g, pl.ds((w % 2) * DW + q * SUB, SUB), :],
                    osem)
        return 0

    lax.fori_loop(0, NWAVE // 2, dpair, 0)
    drain_store(SUB, obA, semOA)
    drain_store(SUB, obB, semOB)


@jax.jit
def kernel(x, in_degree, out_degree, atom_w, in_w, out_w, graph_token):
    x3 = x.astype(jnp.int32).reshape(NG, NN, NF)
    per_g = jnp.concatenate(
        [jnp.zeros((NG, 1, NF), jnp.int32), x3], axis=1).reshape(NG, NR * NF)
    idx9 = jnp.pad(per_g, ((0, 0), (0, GSTRIDE - NR * NF))).reshape(-1)
    ind_f = in_degree.astype(jnp.int32).reshape(NG * NN)
    outd_f = out_degree.astype(jnp.int32).reshape(NG * NN)

    kfn = pl.kernel(
        _body,
        out_type=(
            jax.ShapeDtypeStruct((NG, NR, H), jnp.float32),
            jax.ShapeDtypeStruct((NG, NN, H), jnp.float32),
        ),
        mesh=plsc.VectorSubcoreMesh(core_axis_name="c", subcore_axis_name="s"),
        scratch_types=[
            pltpu.VMEM((GPW * GSTRIDE,), jnp.int32),
            pltpu.VMEM((GPW * NN,), jnp.int32),
            pltpu.VMEM((GPW * NN,), jnp.int32),
            pltpu.VMEM((ROWS, H), jnp.float32),
            pltpu.VMEM((ROWS, H), jnp.float32),
            pltpu.VMEM((SUB, H), jnp.float32),
            pltpu.VMEM((SUB, H), jnp.float32),
            pltpu.VMEM((1, H), jnp.float32),
            pltpu.SemaphoreType.DMA,
            pltpu.SemaphoreType.DMA,
            pltpu.SemaphoreType.DMA,
            pltpu.SemaphoreType.DMA,
        ],
    )
    return kfn(idx9, ind_f, outd_f, atom_w, in_w, out_w, graph_token)


# speedup vs baseline: 1.0076x; 1.0076x over previous
import jax
import jax.numpy as jnp
from jax import lax
from jax.experimental import pallas as pl
from jax.experimental.pallas import tpu as pltpu
from jax.experimental.pallas import tpu_sc as plsc

NG = 256
NN = 64
NR = NN + 1
NF = 9
H = 768
L = 16
NC = 2
NS = 16
NW = NC * NS

GPW = NG // NW
SUB = 8
ROWS = SUB * NF
KPG = 9
GSTRIDE = 592
SMALL = 16
CB = H // L


def _body(idx9, ind_f, outd_f, atom_w, in_w, out_w, tok,
          out3, deg3,
          xidx, din, dout, bufA, bufB, obA, obB, tokv,
          semA, semB, semOA, semOB):
    wid = lax.axis_index("s") * NC + lax.axis_index("c")

    pltpu.sync_copy(idx9.at[pl.ds(wid * GPW * GSTRIDE, GPW * GSTRIDE)], xidx)
    pltpu.sync_copy(ind_f.at[pl.ds(wid * GPW * NN, GPW * NN)], din)
    pltpu.sync_copy(outd_f.at[pl.ds(wid * GPW * NN, GPW * NN)], dout)
    pltpu.sync_copy(tok, tokv)

    def gsize(k):
        return ROWS if k < 8 else SMALL

    def fire(gi, k, buf, sem):
        n = gsize(k)
        pltpu.async_copy(
            atom_w.at[xidx.at[pl.ds(gi * GSTRIDE + k * ROWS, n)]],
            buf.at[pl.ds(0, n)], sem)

    def wait_g(gi, k, buf, sem):
        n = gsize(k)
        pltpu.make_async_copy(
            atom_w.at[xidx.at[pl.ds(gi * GSTRIDE + k * ROWS, n)]],
            buf.at[pl.ds(0, n)], sem).wait()

    def compute(k, buf, ob):
        nrows = SUB if k < 8 else 1

        def jbody(j, _):
            c = j * L
            for n in range(nrows):
                if k == 0 and n == 0:
                    ob[0, pl.ds(c, L)] = tokv[0, pl.ds(c, L)]
                else:
                    base = n * NF
                    acc = buf[base, pl.ds(c, L)]
                    for f in range(1, NF):
                        acc = acc + buf[base + f, pl.ds(c, L)]
                    ob[n, pl.ds(c, L)] = acc
            return 0

        lax.fori_loop(0, CB, jbody, 0)

    def store(gi, k, ob, sem):
        g = wid * GPW + gi
        if k < 8:
            pltpu.async_copy(ob, out3.at[g, pl.ds(k * SUB, SUB), :], sem)
        else:
            pltpu.async_copy(ob.at[pl.ds(0, 1)], out3.at[g, pl.ds(NN, 1), :],
                             sem)

    def drain_store(nrows, ob, sem):
        pltpu.make_async_copy(out3.at[0, pl.ds(0, nrows), :],
                              ob.at[pl.ds(0, nrows)], sem).wait()

    fire(0, 0, bufA, semA)

    def pair_body(gp, _):
        gi0 = gp * 2
        for m in range(2 * KPG):
            gi = gi0 + m // KPG
            k = m % KPG
            buf, sem, ob, osem = ((bufA, semA, obA, semOA) if m % 2 == 0
                                  else (bufB, semB, obB, semOB))
            nbuf, nsem = (bufB, semB) if m % 2 == 0 else (bufA, semA)
            wait_g(gi, k, buf, sem)
            if m == 2 * KPG - 1:
                @pl.when(gi0 + 2 < GPW)
                def _():
                    fire(gi0 + 2, 0, nbuf, nsem)
            elif k == KPG - 1:
                fire(gi + 1, 0, nbuf, nsem)
            else:
                fire(gi, k + 1, nbuf, nsem)
            prev_k = (m - 2) % (2 * KPG) % KPG
            prev_rows = SUB if prev_k < 8 else 1
            if m >= 2:
                drain_store(prev_rows, ob, osem)
            else:
                @pl.when(gp > 0)
                def _():
                    drain_store(prev_rows, ob, osem)
            compute(k, buf, ob)
            store(gi, k, ob, osem)
        return 0

    with jax.named_scope("atom_phase"):
        lax.fori_loop(0, GPW // 2, pair_body, 0)
        drain_store(SUB, obA, semOA)
        drain_store(1, obB, semOB)

    DW = 32
    NWAVE = GPW * NN // DW

    def dfire(w, buf, sem):
        pltpu.async_copy(in_w.at[din.at[pl.ds(w * DW, DW)]],
                         buf.at[pl.ds(0, DW)], sem)
        pltpu.async_copy(out_w.at[dout.at[pl.ds(w * DW, DW)]],
                         buf.at[pl.ds(DW, DW)], sem)

    def dwait(w, buf, sem):
        pltpu.make_async_copy(in_w.at[din.at[pl.ds(w * DW, DW)]],
                              buf.at[pl.ds(0, DW)], sem).wait()
        pltpu.make_async_copy(out_w.at[dout.at[pl.ds(w * DW, DW)]],
                              buf.at[pl.ds(DW, DW)], sem).wait()

    dfire(0, bufA, semA)

    def dpair(wp, _):
        for t in range(2):
            w = wp * 2 + t
            buf, sem = (bufA, semA) if t == 0 else (bufB, semB)
            nbuf, nsem = (bufB, semB) if t == 0 else (bufA, semA)
            dwait(w, buf, sem)
            if t == 0:
                dfire(w + 1, nbuf, nsem)
            else:
                @pl.when(wp + 1 < NWAVE // 2)
                def _():
                    dfire(w + 1, nbuf, nsem)
            for q in range(4):
                ob, osem = ((obA, semOA) if (t * 4 + q) % 2 == 0
                            else (obB, semOB))
                if t == 0 and q < 2:
                    @pl.when(wp > 0)
                    def _():
                        drain_store(SUB, ob, osem)
                else:
                    drain_store(SUB, ob, osem)

                def jbody(j, _):
                    c = j * L
                    for n in range(SUB):
                        r = q * SUB + n
                        ob[n, pl.ds(c, L)] = (buf[r, pl.ds(c, L)]
                                              + buf[DW + r, pl.ds(c, L)])
                    return 0

                lax.fori_loop(0, CB, jbody, 0)
                g = wid * GPW + w // 2
                pltpu.async_copy(
                    ob, deg3.at[g, pl.ds((w % 2) * DW + q * SUB, SUB), :],
                    osem)
        return 0

    with jax.named_scope("degree_phase"):
        lax.fori_loop(0, NWAVE // 2, dpair, 0)
        drain_store(SUB, obA, semOA)
        drain_store(SUB, obB, semOB)


@jax.jit
def kernel(x, in_degree, out_degree, atom_w, in_w, out_w, graph_token):
    x3 = x.astype(jnp.int32).reshape(NG, NN, NF)
    per_g = jnp.concatenate(
        [jnp.zeros((NG, 1, NF), jnp.int32), x3], axis=1).reshape(NG, NR * NF)
    idx9 = jnp.pad(per_g, ((0, 0), (0, GSTRIDE - NR * NF))).reshape(-1)
    ind_f = in_degree.astype(jnp.int32).reshape(NG * NN)
    outd_f = out_degree.astype(jnp.int32).reshape(NG * NN)

    kfn = pl.kernel(
        _body,
        out_type=(
            jax.ShapeDtypeStruct((NG, NR, H), jnp.float32),
            jax.ShapeDtypeStruct((NG, NN, H), jnp.float32),
        ),
        mesh=plsc.VectorSubcoreMesh(core_axis_name="c", subcore_axis_name="s"),
        scratch_types=[
            pltpu.VMEM((GPW * GSTRIDE,), jnp.int32),
            pltpu.VMEM((GPW * NN,), jnp.int32),
            pltpu.VMEM((GPW * NN,), jnp.int32),
            pltpu.VMEM((ROWS, H), jnp.float32),
            pltpu.VMEM((ROWS, H), jnp.float32),
            pltpu.VMEM((SUB, H), jnp.float32),
            pltpu.VMEM((SUB, H), jnp.float32),
            pltpu.VMEM((1, H), jnp.float32),
            pltpu.SemaphoreType.DMA,
            pltpu.SemaphoreType.DMA,
            pltpu.SemaphoreType.DMA,
            pltpu.SemaphoreType.DMA,
        ],
    )
    return kfn(idx9, ind_f, outd_f, atom_w, in_w, out_w, graph_token)
